# baseline (speedup 1.0000x reference)
import functools

import jax
import jax.numpy as jnp
from jax import lax
from jax.experimental import pallas as pl
from jax.experimental.pallas import tpu as pltpu
from jax.experimental.pallas import tpu_sc as plsc

N = 10000
E = 320000
R = 65
F_IN = 128
H1 = 64
H2 = 32
B = 4096

NC = 2
NS = 16
NW = NC * NS
E_PAD = 327680
EPW = E_PAD // NW
EPT = E_PAD // NS
CH = 2048
NB = CH // 128
NCHUNK = EPW // CH
NSB = CH // 64
N_PAD = 10240
NR_PAD = 655360
NR_SL = NR_PAD // NS
ZCH = 5120
RPT = N_PAD // NS

_mesh = plsc.VectorSubcoreMesh(
    core_axis_name="c", subcore_axis_name="s", num_cores=NC, num_subcores=NS)
_sc_params = pltpu.CompilerParams(
    use_tc_tiling_on_sc=False, needs_layout_passes=False)

_f32 = jnp.float32
_bf16 = jnp.bfloat16
_i32 = jnp.int32

def _mk_perm(width):
    perm = [0] * width
    for g in range(width // 32):
        for j in range(16):
            perm[32 * g + 2 * j] = 32 * g + j
            perm[32 * g + 2 * j + 1] = 32 * g + 16 + j
    return perm


_PERM64 = _mk_perm(64)
_PERM32 = _mk_perm(32)


@functools.partial(
    pl.kernel,
    out_type=jax.ShapeDtypeStruct((2, E_PAD), _f32),
    mesh=_mesh,
    compiler_params=_sc_params,
    scratch_types=[
        pltpu.VMEM((CH,), _i32),
        pltpu.VMEM((CH,), _i32),
        pltpu.VMEM((NB, 128), _i32),
        pltpu.VMEM((CH,), _f32),
        pltpu.VMEM((128,), _f32),
        pltpu.VMEM((ZCH,), _f32),
        pltpu.SemaphoreType.DMA,
        pltpu.VMEM_SHARED((NR_PAD,), _f32),
        pltpu.VMEM_SHARED((NR_PAD,), _f32),
    ],
)
def _prep_kernel(dst_h, et_h, et1_h, out_h, dstv, etv, comp, cb, ones,
                 zb, sem, cnt0, cnt1):
    cid = lax.axis_index("c")
    sid = lax.axis_index("s")
    wid = sid * NC + cid

    def _fill(i, _):
        ones[pl.ds(i * 16, 16)] = jnp.ones((16,), _f32)
        return 0
    lax.fori_loop(0, 128 // 16, _fill, 0)

    def _zero(i, _):
        zb[pl.ds(i * 16, 16)] = jnp.zeros((16,), _f32)
        return 0
    lax.fori_loop(0, ZCH // 16, _zero, 0)

    for q in range(NR_SL // ZCH):
        sl = pl.ds(sid * NR_SL + q * ZCH, ZCH)
        pltpu.sync_copy(zb, cnt0.at[sl])
        pltpu.sync_copy(zb, cnt1.at[sl])
    plsc.subcore_barrier()

    def _count_chunk(ci, _):
        base = pl.multiple_of(sid * EPT + ci * CH, 128)
        pltpu.sync_copy(dst_h.at[pl.ds(base, CH)], dstv)
        for eth, cnt_sh in ((et_h, cnt0), (et1_h, cnt1)):
            pltpu.sync_copy(eth.at[pl.ds(base, CH)], etv)

            def _rows(r, _):
                for k in range(8):
                    off = r * 128 + k * 16
                    comp[r, pl.ds(k * 16, 16)] = (
                        dstv[pl.ds(off, 16)] * R + etv[pl.ds(off, 16)])
                return 0
            lax.fori_loop(0, NB, _rows, 0)

            def _scat(b, _):
                pltpu.sync_copy(ones, cnt_sh.at[comp.at[b]], add=True)
                return 0
            lax.fori_loop(0, NB, _scat, 0)
        return 0
    lax.fori_loop(0, EPT // CH, _count_chunk, 0)
    plsc.subcore_barrier()

    for cnt_sh in (cnt0, cnt1):
        for q in range(NR_SL // ZCH):
            sl = pl.ds(sid * NR_SL + q * ZCH, ZCH)
            pltpu.sync_copy(cnt_sh.at[sl], zb)

            def _recip(i, _):
                s2 = pl.ds(i * 16, 16)
                zb[s2] = 1.0 / jnp.maximum(zb[s2], 1.0)
                return 0
            lax.fori_loop(0, ZCH // 16, _recip, 0)
            pltpu.sync_copy(zb, cnt_sh.at[sl])
    plsc.subcore_barrier()

    def _coef_chunk(ci, _):
        base = pl.multiple_of(wid * EPW + ci * CH, 128)
        pltpu.sync_copy(dst_h.at[pl.ds(base, CH)], dstv)
        for k, (eth, cnt_sh) in enumerate(((et_h, cnt0), (et1_h, cnt1))):
            pltpu.sync_copy(eth.at[pl.ds(base, CH)], etv)

            def _rows(r, _):
                for kk in range(8):
                    off = r * 128 + kk * 16
                    comp[r, pl.ds(kk * 16, 16)] = (
                        dstv[pl.ds(off, 16)] * R + etv[pl.ds(off, 16)])
                return 0
            lax.fori_loop(0, NB, _rows, 0)

            def _gath(b, _):
                pltpu.async_copy(cnt_sh.at[comp.at[b]],
                                 cb.at[pl.ds(b * 128, 128)], sem).wait()
                return 0
            lax.fori_loop(0, NB, _gath, 0)
            pltpu.sync_copy(cb, out_h.at[k, pl.ds(base, CH)])
        return 0
    lax.fori_loop(0, NCHUNK, _coef_chunk, 0)


def _make_agg(passes):
    @functools.partial(
        pl.kernel,
        out_type=jax.ShapeDtypeStruct((NC, N_PAD, 128), _f32),
        mesh=_mesh,
        compiler_params=_sc_params,
        scratch_types=[
            pltpu.VMEM((CH,), _i32),
            pltpu.VMEM((CH,), _i32),
            pltpu.VMEM((CH,), _f32),
            pltpu.VMEM((NSB, 64), _i32),
            pltpu.VMEM((CH,), _i32),
            pltpu.VMEM((128, 128), _bf16),
            pltpu.VMEM((64, 128), _f32),
            pltpu.VMEM((128, 128), _f32),
            pltpu.SemaphoreType.DMA,
            pltpu.SemaphoreType.DMA,
            pltpu.VMEM_SHARED((N_PAD, 128), _f32),
        ],
    )
    def _agg(h_h, src_h, dst2_h, et_h, et1_h, c_h, c1_h, out_h,
             srcv, etv, cv, dstv, gidx, rows3, rfp, zb,
             gs0, gs1, acc):
        cid = lax.axis_index("c")
        sid = lax.axis_index("s")
        wid = sid * NC + cid
        gsem = (gs0, gs1)
        rbuf = [rows3.at[pl.ds(p * 64, 64), :] for p in range(2)]

        def _zero(i, _):
            r = i // 8
            q = i % 8
            zb[r, pl.ds(q * 16, 16)] = jnp.zeros((16,), _f32)
            return 0
        lax.fori_loop(0, 128 * 8, _zero, 0)

        def _zero2(i, _):
            r = i // 8
            q = i % 8
            rfp[r, pl.ds(q * 16, 16)] = jnp.zeros((16,), _f32)
            return 0
        lax.fori_loop(0, 64 * 8, _zero2, 0)
        for q in range(RPT // 128):
            pltpu.sync_copy(zb, acc.at[pl.ds(sid * RPT + q * 128, 128), :])
        plsc.subcore_barrier()

        for slot, scale_qs in passes:
            eth = et_h if slot == 0 else et1_h
            ch = c_h if slot == 0 else c1_h

            def _chunk(ci, _):
                base = pl.multiple_of(wid * EPW + ci * CH, 128)
                rbase = pl.multiple_of(wid * (EPW // 64) + ci * NSB, 8)
                pltpu.sync_copy(src_h.at[pl.ds(base, CH)], srcv)
                pltpu.sync_copy(eth.at[pl.ds(base, CH)], etv)
                pltpu.sync_copy(ch.at[pl.ds(base, CH)], cv)
                pltpu.sync_copy(dst2_h.at[pl.ds(rbase, NSB), :], dstv)

                def _gix(j, _):
                    sl = pl.ds(j * 16, 16)
                    gidx[sl] = etv[sl] * N + srcv[sl]
                    return 0
                lax.fori_loop(0, CH // 16, _gix, 0)

                def _do_batch(b, p):
                    pltpu.make_async_copy(
                        h_h.at[gidx.at[pl.ds(b * 64, 64)]],
                        rbuf[p], gsem[p]).wait()

                    def _scale(g, _):
                        cvec16 = cv[pl.ds(b * 64 + g * 16, 16)]
                        for e in range(16):
                            cbr = jnp.full((16,), cvec16[e], _f32)
                            i = p * 64 + g * 16 + e
                            io = g * 16 + e
                            for q in scale_qs:
                                lo, hi = plsc.unpack(
                                    rows3[i, pl.ds(q * 32, 32)],
                                    format=plsc.PackFormat.INTERLEAVED)
                                rfp[io, pl.ds(q * 32, 16)] = lo * cbr
                                rfp[io, pl.ds(q * 32 + 16, 16)] = hi * cbr
                        return 0
                    lax.fori_loop(0, 4, _scale, 0)
                    pltpu.sync_copy(rfp, acc.at[dstv.at[b]], add=True)

                pltpu.async_copy(
                    h_h.at[gidx.at[pl.ds(0, 64)]], rbuf[0], gsem[0])

                def _pair(k, _):
                    b0 = k * 2
                    pltpu.async_copy(
                        h_h.at[gidx.at[pl.ds(b0 * 64 + 64, 64)]],
                        rbuf[1], gsem[1])
                    _do_batch(b0, 0)

                    @pl.when(k < NSB // 2 - 1)
                    def _():
                        pltpu.async_copy(
                            h_h.at[gidx.at[pl.ds(b0 * 64 + 128, 64)]],
                            rbuf[0], gsem[0])
                    _do_batch(b0 + 1, 1)
                    return 0
                lax.fori_loop(0, NSB // 2, _pair, 0)
                return 0
            lax.fori_loop(0, NCHUNK, _chunk, 0)
        plsc.subcore_barrier()

        for q in range(RPT // 128):
            sl = pl.ds(sid * RPT + q * 128, 128)
            pltpu.sync_copy(acc.at[sl, :], zb)
            pltpu.sync_copy(zb, out_h.at[cid, sl, :])
    return _agg


_agg_l1_main = _make_agg(((0, (0, 1, 2, 3)),))
_agg_l1_aug = _make_agg(((1, (0, 1)),))
_agg_l2_main = _make_agg(((0, (0, 1)),))
_agg_l2_aug = _make_agg(((1, (2,)),))


@functools.partial(
    pl.kernel,
    out_type=(jax.ShapeDtypeStruct((B, 128), _f32),
              jax.ShapeDtypeStruct((B, 128), _f32),
              jax.ShapeDtypeStruct((B, 128), _f32),
              jax.ShapeDtypeStruct((B, 128), _f32)),
    mesh=_mesh,
    compiler_params=_sc_params,
    scratch_types=[
        pltpu.VMEM((128,), _i32),
        pltpu.VMEM((128, 128), _f32),
        pltpu.SemaphoreType.DMA,
    ],
)
def _pair_kernel(x1_h, x2_h, i0_h, i1_h, e1a_h, e1b_h, e2a_h, e2b_h,
                 iv, buf, sem):
    cid = lax.axis_index("c")
    sid = lax.axis_index("s")
    wid = sid * NC + cid
    base = pl.multiple_of(wid * (B // NW), 128)
    sl = pl.ds(base, 128)
    pltpu.sync_copy(i0_h.at[sl], iv)
    pltpu.async_copy(x1_h.at[iv], buf, sem).wait()
    pltpu.sync_copy(buf, e1a_h.at[sl, :])
    pltpu.async_copy(x2_h.at[iv], buf, sem).wait()
    pltpu.sync_copy(buf, e1b_h.at[sl, :])
    pltpu.sync_copy(i1_h.at[sl], iv)
    pltpu.async_copy(x1_h.at[iv], buf, sem).wait()
    pltpu.sync_copy(buf, e2a_h.at[sl, :])
    pltpu.async_copy(x2_h.at[iv], buf, sem).wait()
    pltpu.sync_copy(buf, e2b_h.at[sl, :])


def _h1_body(xo_ref, xa_ref, w_ref, o_ref):
    o_ref[0] = jnp.concatenate(
        [jnp.dot(xo_ref[...], w_ref[0], preferred_element_type=_f32),
         jnp.dot(xa_ref[...], w_ref[0], preferred_element_type=_f32)],
        axis=1).astype(_bf16)


_h1_call = pl.pallas_call(
    _h1_body,
    grid=(R,),
    in_specs=[pl.BlockSpec((N, F_IN), lambda r: (0, 0)),
              pl.BlockSpec((N, F_IN), lambda r: (0, 0)),
              pl.BlockSpec((1, F_IN, H1), lambda r: (r, 0, 0))],
    out_specs=pl.BlockSpec((1, N, 128), lambda r: (r, 0, 0)),
    out_shape=jax.ShapeDtypeStruct((R, N, 128), _bf16),
)


def _h2_body(xo_ref, xa_ref, xaa_ref, w_ref, o_ref):
    parts = [jnp.dot(x_ref[:, 0:H1], w_ref[0], preferred_element_type=_f32)
             for x_ref in (xo_ref, xa_ref, xaa_ref)]
    parts.append(jnp.zeros((N, H2), _f32))
    o_ref[0] = jnp.concatenate(parts, axis=1).astype(_bf16)


_h2_call = pl.pallas_call(
    _h2_body,
    grid=(R,),
    in_specs=[pl.BlockSpec((N, 128), lambda r: (0, 0)),
              pl.BlockSpec((N, 128), lambda r: (0, 0)),
              pl.BlockSpec((N, 128), lambda r: (0, 0)),
              pl.BlockSpec((1, H1, H2), lambda r: (r, 0, 0))],
    out_specs=pl.BlockSpec((1, N, 128), lambda r: (r, 0, 0)),
    out_shape=jax.ShapeDtypeStruct((R, N, 128), _bf16),
)


def _make_combine(F, H, col, relu):
    def body(a_ref, x_ref, root_ref, b_ref, o_ref):
        a = a_ref[0, :, col * H:(col + 1) * H] + a_ref[1, :, col * H:(col + 1) * H]
        y = (a
             + jnp.dot(x_ref[:, 0:F], root_ref[...],
                       preferred_element_type=_f32)
             + b_ref[...])
        if relu:
            y = jnp.maximum(y, 0.0)
        pad = jnp.zeros((y.shape[0], 128 - H), _f32)
        o_ref[...] = jnp.concatenate([y, pad], axis=1)

    BN = 2000
    return pl.pallas_call(
        body,
        grid=(N // BN,),
        in_specs=[pl.BlockSpec((NC, BN, 128), lambda i: (0, i, 0)),
                  pl.BlockSpec((BN, F_IN), lambda i: (i, 0)),
                  pl.BlockSpec((F, H), lambda i: (0, 0)),
                  pl.BlockSpec((1, H), lambda i: (0, 0))],
        out_specs=pl.BlockSpec((BN, 128), lambda i: (i, 0)),
        out_shape=jax.ShapeDtypeStruct((N, 128), _f32),
    )


_comb1_0 = _make_combine(F_IN, H1, 0, True)
_comb1_1 = _make_combine(F_IN, H1, 1, True)
_comb2_0 = _make_combine(H1, H2, 0, False)
_comb2_1 = _make_combine(H1, H2, 1, False)
_comb2_2 = _make_combine(H1, H2, 2, False)


def _epi_body(x2o_ref, x2a_ref, x2aa_ref, e1a_ref, e1b_ref, e2a_ref, e2b_ref,
              attt_ref, wb_ref, bb_ref, wc_ref, bc_ref,
              log_ref, ro_ref, roa_ref, x2_ref):
    x2o = x2o_ref[:, 0:H2]
    x2_ref[...] = x2o
    hos = jax.nn.sigmoid(jnp.mean(x2o, axis=0, keepdims=True))
    wb = wb_ref[...]
    bb = bb_ref[0, 0]
    sc1 = jnp.sum(jnp.dot(x2o, wb, preferred_element_type=_f32) * hos,
                  axis=1, keepdims=True) + bb
    sc2 = jnp.sum(jnp.dot(x2a_ref[:, 0:H2], wb,
                          preferred_element_type=_f32) * hos,
                  axis=1, keepdims=True) + bb
    sc2a = jnp.sum(jnp.dot(x2aa_ref[:, 0:H2], wb,
                           preferred_element_type=_f32) * hos,
                   axis=1, keepdims=True) + bb
    ro_ref[...] = jnp.concatenate([sc1, sc2], axis=1)
    roa_ref[...] = jnp.concatenate([sc1, sc2a], axis=1)
    a0 = attt_ref[0, 0]
    a1 = attt_ref[0, 1]
    log = (a0 * jnp.dot(e1a_ref[:, 0:H1], wc_ref[0:H1, :],
                        preferred_element_type=_f32)
           + a1 * jnp.dot(e1b_ref[:, 0:H2], wc_ref[H1:H1 + H2, :],
                          preferred_element_type=_f32)
           + a0 * jnp.dot(e2a_ref[:, 0:H1], wc_ref[H1 + H2:2 * H1 + H2, :],
                          preferred_element_type=_f32)
           + a1 * jnp.dot(e2b_ref[:, 0:H2], wc_ref[2 * H1 + H2:, :],
                          preferred_element_type=_f32)
           + bc_ref[...])
    log_ref[...] = log


_epi_call = pl.pallas_call(
    _epi_body,
    out_shape=(jax.ShapeDtypeStruct((B, R), _f32),
               jax.ShapeDtypeStruct((N, 2), _f32),
               jax.ShapeDtypeStruct((N, 2), _f32),
               jax.ShapeDtypeStruct((N, H2), _f32)),
)


def kernel(x_o, x_a, edge_index, edge_type, edge_type1, idx, W1, root1, b1,
           W2, root2, b2, attt, Wb, bb, Wc, bc):
    src = edge_index[0]
    dst = edge_index[1]
    pad = E_PAD - E
    zpad = jnp.zeros((pad,), _i32)
    src_p = jnp.concatenate([src, zpad])
    dst_p = jnp.concatenate([dst, jnp.full((pad,), N, _i32)])
    et_p = jnp.concatenate([edge_type, zpad])
    et1_p = jnp.concatenate([edge_type1, zpad])
    dst2 = dst_p.reshape(E_PAD // 64, 64)

    c_all = _prep_kernel(dst_p, et_p, et1_p)
    c_et = c_all[0]
    c_et1 = c_all[1]

    w1p = W1[:, :, _PERM64].astype(_bf16)
    h1 = _h1_call(x_o.astype(_bf16), x_a.astype(_bf16),
                  w1p).reshape(R * N, 128)
    agg_oa = _agg_l1_main(h1, src_p, dst2, et_p, et1_p, c_et, c_et1)
    agg_aa = _agg_l1_aug(h1, src_p, dst2, et_p, et1_p, c_et, c_et1)
    b1r = b1.reshape(1, H1)
    x1_o = _comb1_0(agg_oa, x_o, root1, b1r)
    x1_a = _comb1_1(agg_oa, x_a, root1, b1r)
    x1_aa = _comb1_0(agg_aa, x_o, root1, b1r)

    w2p = W2[:, :, _PERM32].astype(_bf16)
    h2 = _h2_call(x1_o.astype(_bf16), x1_a.astype(_bf16),
                  x1_aa.astype(_bf16), w2p).reshape(R * N, 128)
    agg2 = _agg_l2_main(h2, src_p, dst2, et_p, et1_p, c_et, c_et1)
    agg2_aug = _agg_l2_aug(h2, src_p, dst2, et_p, et1_p, c_et, c_et1)
    b2r = b2.reshape(1, H2)
    x2_o = _comb2_0(agg2, x1_o, root2, b2r)
    x2_a = _comb2_1(agg2, x1_a, root2, b2r)
    x2_aa = _comb2_2(agg2_aug, x1_aa, root2, b2r)

    e1a, e1b, e2a, e2b = _pair_kernel(x1_o, x2_o, idx[0], idx[1])
    log, ret_os, ret_os_a, x2_exact = _epi_call(
        x2_o, x2_a, x2_aa, e1a, e1b, e2a, e2b,
        attt.reshape(1, 2), Wb[0], bb.reshape(1, 1), Wc, bc.reshape(1, R))
    return (log, ret_os, ret_os_a, x2_exact)


# speedup vs baseline: 1.1068x; 1.1068x over previous
import functools

import jax
import jax.numpy as jnp
from jax import lax
from jax.experimental import pallas as pl
from jax.experimental.pallas import tpu as pltpu
from jax.experimental.pallas import tpu_sc as plsc

N = 10000
E = 320000
R = 65
F_IN = 128
H1 = 64
H2 = 32
B = 4096

NC = 2
NS = 16
NW = NC * NS
E_PAD = 327680
EPW = E_PAD // NW
EPT = E_PAD // NS
CH = 2048
NB = CH // 128
NCHUNK = EPW // CH
NSB = CH // 64
N_PAD = 10240
NR_PAD = 655360
NR_SL = NR_PAD // NS
ZCH = 5120
RPT = N_PAD // NS

_mesh = plsc.VectorSubcoreMesh(
    core_axis_name="c", subcore_axis_name="s", num_cores=NC, num_subcores=NS)
_sc_params = pltpu.CompilerParams(
    use_tc_tiling_on_sc=False, needs_layout_passes=False)

_f32 = jnp.float32
_bf16 = jnp.bfloat16
_i32 = jnp.int32

def _mk_perm(width):
    perm = [0] * width
    for g in range(width // 32):
        for j in range(16):
            perm[32 * g + 2 * j] = 32 * g + j
            perm[32 * g + 2 * j + 1] = 32 * g + 16 + j
    return perm


_PERM64 = _mk_perm(64)
_PERM32 = _mk_perm(32)


@functools.partial(
    pl.kernel,
    out_type=jax.ShapeDtypeStruct((2, E_PAD), _f32),
    mesh=_mesh,
    compiler_params=_sc_params,
    scratch_types=[
        pltpu.VMEM((CH,), _i32),
        pltpu.VMEM((CH,), _i32),
        pltpu.VMEM((NB, 128), _i32),
        pltpu.VMEM((CH,), _f32),
        pltpu.VMEM((128,), _f32),
        pltpu.VMEM((ZCH,), _f32),
        pltpu.SemaphoreType.DMA,
        pltpu.VMEM_SHARED((NR_PAD,), _f32),
        pltpu.VMEM_SHARED((NR_PAD,), _f32),
    ],
)
def _prep_kernel(dst_h, et_h, et1_h, out_h, dstv, etv, comp, cb, ones,
                 zb, sem, cnt0, cnt1):
    cid = lax.axis_index("c")
    sid = lax.axis_index("s")
    wid = sid * NC + cid

    def _fill(i, _):
        ones[pl.ds(i * 16, 16)] = jnp.ones((16,), _f32)
        return 0
    lax.fori_loop(0, 128 // 16, _fill, 0)

    def _zero(i, _):
        zb[pl.ds(i * 16, 16)] = jnp.zeros((16,), _f32)
        return 0
    lax.fori_loop(0, ZCH // 16, _zero, 0)

    for q in range(NR_SL // ZCH):
        sl = pl.ds(sid * NR_SL + q * ZCH, ZCH)
        pltpu.sync_copy(zb, cnt0.at[sl])
        pltpu.sync_copy(zb, cnt1.at[sl])
    plsc.subcore_barrier()

    def _count_chunk(ci, _):
        base = pl.multiple_of(sid * EPT + ci * CH, 128)
        pltpu.sync_copy(dst_h.at[pl.ds(base, CH)], dstv)
        for eth, cnt_sh in ((et_h, cnt0), (et1_h, cnt1)):
            pltpu.sync_copy(eth.at[pl.ds(base, CH)], etv)

            def _rows(r, _):
                for k in range(8):
                    off = r * 128 + k * 16
                    comp[r, pl.ds(k * 16, 16)] = (
                        dstv[pl.ds(off, 16)] * R + etv[pl.ds(off, 16)])
                return 0
            lax.fori_loop(0, NB, _rows, 0)

            def _scat(b, _):
                pltpu.sync_copy(ones, cnt_sh.at[comp.at[b]], add=True)
                return 0
            lax.fori_loop(0, NB, _scat, 0)
        return 0
    lax.fori_loop(0, EPT // CH, _count_chunk, 0)
    plsc.subcore_barrier()

    for cnt_sh in (cnt0, cnt1):
        for q in range(NR_SL // ZCH):
            sl = pl.ds(sid * NR_SL + q * ZCH, ZCH)
            pltpu.sync_copy(cnt_sh.at[sl], zb)

            def _recip(i, _):
                s2 = pl.ds(i * 16, 16)
                zb[s2] = 1.0 / jnp.maximum(zb[s2], 1.0)
                return 0
            lax.fori_loop(0, ZCH // 16, _recip, 0)
            pltpu.sync_copy(zb, cnt_sh.at[sl])
    plsc.subcore_barrier()

    def _coef_chunk(ci, _):
        base = pl.multiple_of(wid * EPW + ci * CH, 128)
        pltpu.sync_copy(dst_h.at[pl.ds(base, CH)], dstv)
        for k, (eth, cnt_sh) in enumerate(((et_h, cnt0), (et1_h, cnt1))):
            pltpu.sync_copy(eth.at[pl.ds(base, CH)], etv)

            def _rows(r, _):
                for kk in range(8):
                    off = r * 128 + kk * 16
                    comp[r, pl.ds(kk * 16, 16)] = (
                        dstv[pl.ds(off, 16)] * R + etv[pl.ds(off, 16)])
                return 0
            lax.fori_loop(0, NB, _rows, 0)

            def _gath(b, _):
                pltpu.async_copy(cnt_sh.at[comp.at[b]],
                                 cb.at[pl.ds(b * 128, 128)], sem).wait()
                return 0
            lax.fori_loop(0, NB, _gath, 0)
            pltpu.sync_copy(cb, out_h.at[k, pl.ds(base, CH)])
        return 0
    lax.fori_loop(0, NCHUNK, _coef_chunk, 0)


def _make_agg(passes):
    @functools.partial(
        pl.kernel,
        out_type=jax.ShapeDtypeStruct((NC, N_PAD, 128), _f32),
        mesh=_mesh,
        compiler_params=_sc_params,
        scratch_types=[
            pltpu.VMEM((CH,), _i32),
            pltpu.VMEM((CH,), _i32),
            pltpu.VMEM((CH,), _f32),
            pltpu.VMEM((NSB, 64), _i32),
            pltpu.VMEM((CH,), _i32),
            pltpu.VMEM((128, 128), _bf16),
            pltpu.VMEM((64, 128), _f32),
            pltpu.VMEM((128, 128), _f32),
            pltpu.SemaphoreType.DMA,
            pltpu.SemaphoreType.DMA,
            pltpu.VMEM_SHARED((N_PAD, 128), _f32),
        ],
    )
    def _agg(h_h, src_h, dst2_h, et_h, et1_h, c_h, c1_h, out_h,
             srcv, etv, cv, dstv, gidx, rows3, rfp, zb,
             gs0, gs1, acc):
        cid = lax.axis_index("c")
        sid = lax.axis_index("s")
        wid = sid * NC + cid
        gsem = (gs0, gs1)
        rbuf = [rows3.at[pl.ds(p * 64, 64), :] for p in range(2)]

        def _zero(i, _):
            r = i // 8
            q = i % 8
            zb[r, pl.ds(q * 16, 16)] = jnp.zeros((16,), _f32)
            return 0
        lax.fori_loop(0, 128 * 8, _zero, 0)

        def _zero2(i, _):
            r = i // 8
            q = i % 8
            rfp[r, pl.ds(q * 16, 16)] = jnp.zeros((16,), _f32)
            return 0
        lax.fori_loop(0, 64 * 8, _zero2, 0)
        for q in range(RPT // 128):
            pltpu.sync_copy(zb, acc.at[pl.ds(sid * RPT + q * 128, 128), :])
        plsc.subcore_barrier()

        for slot, scale_qs in passes:
            eth = et_h if slot == 0 else et1_h
            ch = c_h if slot == 0 else c1_h

            def _chunk(ci, _):
                base = pl.multiple_of(wid * EPW + ci * CH, 128)
                rbase = pl.multiple_of(wid * (EPW // 64) + ci * NSB, 8)
                pltpu.sync_copy(src_h.at[pl.ds(base, CH)], srcv)
                pltpu.sync_copy(eth.at[pl.ds(base, CH)], etv)
                pltpu.sync_copy(ch.at[pl.ds(base, CH)], cv)
                pltpu.sync_copy(dst2_h.at[pl.ds(rbase, NSB), :], dstv)

                def _gix(j, _):
                    sl = pl.ds(j * 16, 16)
                    gidx[sl] = etv[sl] * N + srcv[sl]
                    return 0
                lax.fori_loop(0, CH // 16, _gix, 0)

                def _do_batch(b, p):
                    pltpu.make_async_copy(
                        h_h.at[gidx.at[pl.ds(b * 64, 64)]],
                        rbuf[p], gsem[p]).wait()

                    def _scale(g, _):
                        cvec16 = cv[pl.ds(b * 64 + g * 16, 16)]
                        for e in range(16):
                            cbr = jnp.full((16,), cvec16[e], _f32)
                            i = p * 64 + g * 16 + e
                            io = g * 16 + e
                            for q in scale_qs:
                                lo, hi = plsc.unpack(
                                    rows3[i, pl.ds(q * 32, 32)],
                                    format=plsc.PackFormat.INTERLEAVED)
                                rfp[io, pl.ds(q * 32, 16)] = lo * cbr
                                rfp[io, pl.ds(q * 32 + 16, 16)] = hi * cbr
                        return 0
                    lax.fori_loop(0, 4, _scale, 0)
                    pltpu.sync_copy(rfp, acc.at[dstv.at[b]], add=True)

                pltpu.async_copy(
                    h_h.at[gidx.at[pl.ds(0, 64)]], rbuf[0], gsem[0])

                def _pair(k, _):
                    b0 = k * 2
                    pltpu.async_copy(
                        h_h.at[gidx.at[pl.ds(b0 * 64 + 64, 64)]],
                        rbuf[1], gsem[1])
                    _do_batch(b0, 0)

                    @pl.when(k < NSB // 2 - 1)
                    def _():
                        pltpu.async_copy(
                            h_h.at[gidx.at[pl.ds(b0 * 64 + 128, 64)]],
                            rbuf[0], gsem[0])
                    _do_batch(b0 + 1, 1)
                    return 0
                lax.fori_loop(0, NSB // 2, _pair, 0)
                return 0
            lax.fori_loop(0, NCHUNK, _chunk, 0)
        plsc.subcore_barrier()

        for q in range(RPT // 128):
            sl = pl.ds(sid * RPT + q * 128, 128)
            pltpu.sync_copy(acc.at[sl, :], zb)
            pltpu.sync_copy(zb, out_h.at[cid, sl, :])
    return _agg


_agg_l1_main = _make_agg(((0, (0, 1, 2, 3)),))
_agg_l1_aug = _make_agg(((1, (0, 1)),))
_agg_l2_main = _make_agg(((0, (0, 1)),))
_agg_l2_aug = _make_agg(((1, (2,)),))


@functools.partial(
    pl.kernel,
    out_type=(jax.ShapeDtypeStruct((B, 128), _f32),
              jax.ShapeDtypeStruct((B, 128), _f32),
              jax.ShapeDtypeStruct((B, 128), _f32),
              jax.ShapeDtypeStruct((B, 128), _f32)),
    mesh=_mesh,
    compiler_params=_sc_params,
    scratch_types=[
        pltpu.VMEM((128,), _i32),
        pltpu.VMEM((128, 128), _f32),
        pltpu.SemaphoreType.DMA,
    ],
)
def _pair_kernel(x1_h, x2_h, i0_h, i1_h, e1a_h, e1b_h, e2a_h, e2b_h,
                 iv, buf, sem):
    cid = lax.axis_index("c")
    sid = lax.axis_index("s")
    wid = sid * NC + cid
    base = pl.multiple_of(wid * (B // NW), 128)
    sl = pl.ds(base, 128)
    pltpu.sync_copy(i0_h.at[sl], iv)
    pltpu.async_copy(x1_h.at[iv], buf, sem).wait()
    pltpu.sync_copy(buf, e1a_h.at[sl, :])
    pltpu.async_copy(x2_h.at[iv], buf, sem).wait()
    pltpu.sync_copy(buf, e1b_h.at[sl, :])
    pltpu.sync_copy(i1_h.at[sl], iv)
    pltpu.async_copy(x1_h.at[iv], buf, sem).wait()
    pltpu.sync_copy(buf, e2a_h.at[sl, :])
    pltpu.async_copy(x2_h.at[iv], buf, sem).wait()
    pltpu.sync_copy(buf, e2b_h.at[sl, :])


def _h1_body(xo_ref, xa_ref, w_ref, o_ref):
    o_ref[...] = jnp.concatenate(
        [jnp.dot(xo_ref[...], w_ref[0], preferred_element_type=_f32),
         jnp.dot(xa_ref[...], w_ref[0], preferred_element_type=_f32)],
        axis=1).astype(_bf16)


_h1_call = pl.pallas_call(
    _h1_body,
    grid=(R,),
    in_specs=[pl.BlockSpec((N, F_IN), lambda r: (0, 0)),
              pl.BlockSpec((N, F_IN), lambda r: (0, 0)),
              pl.BlockSpec((1, F_IN, H1), lambda r: (r, 0, 0))],
    out_specs=pl.BlockSpec((N, 128), lambda r: (r, 0)),
    out_shape=jax.ShapeDtypeStruct((R * N, 128), _bf16),
)


def _h2_body(xo_ref, xa_ref, xaa_ref, w_ref, o_ref):
    parts = [jnp.dot(x_ref[:, 0:H1], w_ref[0], preferred_element_type=_f32)
             for x_ref in (xo_ref, xa_ref, xaa_ref)]
    parts.append(jnp.zeros((N, H2), _f32))
    o_ref[...] = jnp.concatenate(parts, axis=1).astype(_bf16)


_h2_call = pl.pallas_call(
    _h2_body,
    grid=(R,),
    in_specs=[pl.BlockSpec((N, 128), lambda r: (0, 0)),
              pl.BlockSpec((N, 128), lambda r: (0, 0)),
              pl.BlockSpec((N, 128), lambda r: (0, 0)),
              pl.BlockSpec((1, H1, H2), lambda r: (r, 0, 0))],
    out_specs=pl.BlockSpec((N, 128), lambda r: (r, 0)),
    out_shape=jax.ShapeDtypeStruct((R * N, 128), _bf16),
)


def _make_combine(F, H, col, relu):
    def body(a_ref, x_ref, root_ref, b_ref, o_ref):
        a = a_ref[0, :, col * H:(col + 1) * H] + a_ref[1, :, col * H:(col + 1) * H]
        y = (a
             + jnp.dot(x_ref[:, 0:F], root_ref[...],
                       preferred_element_type=_f32)
             + b_ref[...])
        if relu:
            y = jnp.maximum(y, 0.0)
        pad = jnp.zeros((y.shape[0], 128 - H), _f32)
        o_ref[...] = jnp.concatenate([y, pad], axis=1)

    BN = 2000
    return pl.pallas_call(
        body,
        grid=(N // BN,),
        in_specs=[pl.BlockSpec((NC, BN, 128), lambda i: (0, i, 0)),
                  pl.BlockSpec((BN, F_IN), lambda i: (i, 0)),
                  pl.BlockSpec((F, H), lambda i: (0, 0)),
                  pl.BlockSpec((1, H), lambda i: (0, 0))],
        out_specs=pl.BlockSpec((BN, 128), lambda i: (i, 0)),
        out_shape=jax.ShapeDtypeStruct((N, 128), _f32),
    )


_comb1_0 = _make_combine(F_IN, H1, 0, True)
_comb1_1 = _make_combine(F_IN, H1, 1, True)
_comb2_0 = _make_combine(H1, H2, 0, False)
_comb2_1 = _make_combine(H1, H2, 1, False)
_comb2_2 = _make_combine(H1, H2, 2, False)


def _epi_body(x2o_ref, x2a_ref, x2aa_ref, e1a_ref, e1b_ref, e2a_ref, e2b_ref,
              attt_ref, wb_ref, bb_ref, wc_ref, bc_ref,
              log_ref, ro_ref, roa_ref, x2_ref):
    x2o = x2o_ref[:, 0:H2]
    x2_ref[...] = x2o
    hos = jax.nn.sigmoid(jnp.mean(x2o, axis=0, keepdims=True))
    wb = wb_ref[...]
    bb = bb_ref[0, 0]
    sc1 = jnp.sum(jnp.dot(x2o, wb, preferred_element_type=_f32) * hos,
                  axis=1, keepdims=True) + bb
    sc2 = jnp.sum(jnp.dot(x2a_ref[:, 0:H2], wb,
                          preferred_element_type=_f32) * hos,
                  axis=1, keepdims=True) + bb
    sc2a = jnp.sum(jnp.dot(x2aa_ref[:, 0:H2], wb,
                           preferred_element_type=_f32) * hos,
                   axis=1, keepdims=True) + bb
    ro_ref[...] = jnp.concatenate([sc1, sc2], axis=1)
    roa_ref[...] = jnp.concatenate([sc1, sc2a], axis=1)
    a0 = attt_ref[0, 0]
    a1 = attt_ref[0, 1]
    log = (a0 * jnp.dot(e1a_ref[:, 0:H1], wc_ref[0:H1, :],
                        preferred_element_type=_f32)
           + a1 * jnp.dot(e1b_ref[:, 0:H2], wc_ref[H1:H1 + H2, :],
                          preferred_element_type=_f32)
           + a0 * jnp.dot(e2a_ref[:, 0:H1], wc_ref[H1 + H2:2 * H1 + H2, :],
                          preferred_element_type=_f32)
           + a1 * jnp.dot(e2b_ref[:, 0:H2], wc_ref[2 * H1 + H2:, :],
                          preferred_element_type=_f32)
           + bc_ref[...])
    log_ref[...] = log


_epi_call = pl.pallas_call(
    _epi_body,
    out_shape=(jax.ShapeDtypeStruct((B, R), _f32),
               jax.ShapeDtypeStruct((N, 2), _f32),
               jax.ShapeDtypeStruct((N, 2), _f32),
               jax.ShapeDtypeStruct((N, H2), _f32)),
)


def kernel(x_o, x_a, edge_index, edge_type, edge_type1, idx, W1, root1, b1,
           W2, root2, b2, attt, Wb, bb, Wc, bc):
    src = edge_index[0]
    dst = edge_index[1]
    pad = E_PAD - E
    zpad = jnp.zeros((pad,), _i32)
    src_p = jnp.concatenate([src, zpad])
    dst_p = jnp.concatenate([dst, jnp.full((pad,), N, _i32)])
    et_p = jnp.concatenate([edge_type, zpad])
    et1_p = jnp.concatenate([edge_type1, zpad])
    dst2 = dst_p.reshape(E_PAD // 64, 64)

    c_all = _prep_kernel(dst_p, et_p, et1_p)
    c_et = c_all[0]
    c_et1 = c_all[1]

    h1 = _h1_call(x_o, x_a, W1[:, :, _PERM64])
    agg_oa = _agg_l1_main(h1, src_p, dst2, et_p, et1_p, c_et, c_et1)
    agg_aa = _agg_l1_aug(h1, src_p, dst2, et_p, et1_p, c_et, c_et1)
    b1r = b1.reshape(1, H1)
    x1_o = _comb1_0(agg_oa, x_o, root1, b1r)
    x1_a = _comb1_1(agg_oa, x_a, root1, b1r)
    x1_aa = _comb1_0(agg_aa, x_o, root1, b1r)

    h2 = _h2_call(x1_o, x1_a, x1_aa, W2[:, :, _PERM32])
    agg2 = _agg_l2_main(h2, src_p, dst2, et_p, et1_p, c_et, c_et1)
    agg2_aug = _agg_l2_aug(h2, src_p, dst2, et_p, et1_p, c_et, c_et1)
    b2r = b2.reshape(1, H2)
    x2_o = _comb2_0(agg2, x1_o, root2, b2r)
    x2_a = _comb2_1(agg2, x1_a, root2, b2r)
    x2_aa = _comb2_2(agg2_aug, x1_aa, root2, b2r)

    e1a, e1b, e2a, e2b = _pair_kernel(x1_o, x2_o, idx[0], idx[1])
    log, ret_os, ret_os_a, x2_exact = _epi_call(
        x2_o, x2_a, x2_aa, e1a, e1b, e2a, e2b,
        attt.reshape(1, 2), Wb[0], bb.reshape(1, 1), Wc, bc.reshape(1, R))
    return (log, ret_os, ret_os_a, x2_exact)


# speedup vs baseline: 1.1901x; 1.0753x over previous
import functools

import jax
import jax.numpy as jnp
from jax import lax
from jax.experimental import pallas as pl
from jax.experimental.pallas import tpu as pltpu
from jax.experimental.pallas import tpu_sc as plsc

N = 10000
E = 320000
R = 65
F_IN = 128
H1 = 64
H2 = 32
B = 4096

NC = 2
NS = 16
NW = NC * NS
E_PAD = 327680
EPW = E_PAD // NW
EPT = E_PAD // NS
CH = 2048
NB = CH // 128
NCHUNK = EPW // CH
BS = 32
NSB = CH // BS
N_PAD = 10240
NR_PAD = 655360
NR_SL = NR_PAD // NS
ZCH = 5120
RPT = N_PAD // NS

_mesh = plsc.VectorSubcoreMesh(
    core_axis_name="c", subcore_axis_name="s", num_cores=NC, num_subcores=NS)
_sc_params = pltpu.CompilerParams(
    use_tc_tiling_on_sc=False, needs_layout_passes=False)

_f32 = jnp.float32
_bf16 = jnp.bfloat16
_i32 = jnp.int32

def _mk_perm(width):
    perm = [0] * width
    for g in range(width // 32):
        for j in range(16):
            perm[32 * g + 2 * j] = 32 * g + j
            perm[32 * g + 2 * j + 1] = 32 * g + 16 + j
    return perm


_PERM64 = _mk_perm(64)
_PERM32 = _mk_perm(32)


@functools.partial(
    pl.kernel,
    out_type=jax.ShapeDtypeStruct((2, E_PAD), _f32),
    mesh=_mesh,
    compiler_params=_sc_params,
    scratch_types=[
        pltpu.VMEM((CH,), _i32),
        pltpu.VMEM((CH,), _i32),
        pltpu.VMEM((NB, 128), _i32),
        pltpu.VMEM((CH,), _f32),
        pltpu.VMEM((128,), _f32),
        pltpu.VMEM((ZCH,), _f32),
        pltpu.SemaphoreType.DMA,
        pltpu.VMEM_SHARED((NR_PAD,), _f32),
        pltpu.VMEM_SHARED((NR_PAD,), _f32),
    ],
)
def _prep_kernel(dst_h, et_h, et1_h, out_h, dstv, etv, comp, cb, ones,
                 zb, sem, cnt0, cnt1):
    cid = lax.axis_index("c")
    sid = lax.axis_index("s")
    wid = sid * NC + cid

    def _fill(i, _):
        ones[pl.ds(i * 16, 16)] = jnp.ones((16,), _f32)
        return 0
    lax.fori_loop(0, 128 // 16, _fill, 0)

    def _zero(i, _):
        zb[pl.ds(i * 16, 16)] = jnp.zeros((16,), _f32)
        return 0
    lax.fori_loop(0, ZCH // 16, _zero, 0)

    for q in range(NR_SL // ZCH):
        sl = pl.ds(sid * NR_SL + q * ZCH, ZCH)
        pltpu.sync_copy(zb, cnt0.at[sl])
        pltpu.sync_copy(zb, cnt1.at[sl])
    plsc.subcore_barrier()

    def _count_chunk(ci, _):
        base = pl.multiple_of(sid * EPT + ci * CH, 128)
        pltpu.sync_copy(dst_h.at[pl.ds(base, CH)], dstv)
        for eth, cnt_sh in ((et_h, cnt0), (et1_h, cnt1)):
            pltpu.sync_copy(eth.at[pl.ds(base, CH)], etv)

            def _rows(r, _):
                for k in range(8):
                    off = r * 128 + k * 16
                    comp[r, pl.ds(k * 16, 16)] = (
                        dstv[pl.ds(off, 16)] * R + etv[pl.ds(off, 16)])
                return 0
            lax.fori_loop(0, NB, _rows, 0)

            def _scat(b, _):
                pltpu.sync_copy(ones, cnt_sh.at[comp.at[b]], add=True)
                return 0
            lax.fori_loop(0, NB, _scat, 0)
        return 0
    lax.fori_loop(0, EPT // CH, _count_chunk, 0)
    plsc.subcore_barrier()

    for cnt_sh in (cnt0, cnt1):
        for q in range(NR_SL // ZCH):
            sl = pl.ds(sid * NR_SL + q * ZCH, ZCH)
            pltpu.sync_copy(cnt_sh.at[sl], zb)

            def _recip(i, _):
                s2 = pl.ds(i * 16, 16)
                zb[s2] = 1.0 / jnp.maximum(zb[s2], 1.0)
                return 0
            lax.fori_loop(0, ZCH // 16, _recip, 0)
            pltpu.sync_copy(zb, cnt_sh.at[sl])
    plsc.subcore_barrier()

    def _coef_chunk(ci, _):
        base = pl.multiple_of(wid * EPW + ci * CH, 128)
        pltpu.sync_copy(dst_h.at[pl.ds(base, CH)], dstv)
        for k, (eth, cnt_sh) in enumerate(((et_h, cnt0), (et1_h, cnt1))):
            pltpu.sync_copy(eth.at[pl.ds(base, CH)], etv)

            def _rows(r, _):
                for kk in range(8):
                    off = r * 128 + kk * 16
                    comp[r, pl.ds(kk * 16, 16)] = (
                        dstv[pl.ds(off, 16)] * R + etv[pl.ds(off, 16)])
                return 0
            lax.fori_loop(0, NB, _rows, 0)

            def _gath(b, _):
                pltpu.async_copy(cnt_sh.at[comp.at[b]],
                                 cb.at[pl.ds(b * 128, 128)], sem).wait()
                return 0
            lax.fori_loop(0, NB, _gath, 0)
            pltpu.sync_copy(cb, out_h.at[k, pl.ds(base, CH)])
        return 0
    lax.fori_loop(0, NCHUNK, _coef_chunk, 0)


def _make_agg(passes):
    @functools.partial(
        pl.kernel,
        out_type=jax.ShapeDtypeStruct((NC, N_PAD, 128), _f32),
        mesh=_mesh,
        compiler_params=_sc_params,
        scratch_types=[
            pltpu.VMEM((CH,), _i32),
            pltpu.VMEM((CH,), _i32),
            pltpu.VMEM((CH,), _f32),
            pltpu.VMEM((NSB, BS), _i32),
            pltpu.VMEM((CH,), _i32),
            pltpu.VMEM((2 * BS, 128), _f32),
            pltpu.VMEM((BS, 128), _f32),
            pltpu.VMEM((128, 128), _f32),
            pltpu.SemaphoreType.DMA,
            pltpu.SemaphoreType.DMA,
            pltpu.VMEM_SHARED((N_PAD, 128), _f32),
        ],
    )
    def _agg(h_h, src_h, dst2_h, et_h, et1_h, c_h, c1_h, out_h,
             srcv, etv, cv, dstv, gidx, rows3, rfp, zb,
             gs0, gs1, acc):
        cid = lax.axis_index("c")
        sid = lax.axis_index("s")
        wid = sid * NC + cid
        gsem = (gs0, gs1)
        rbuf = [rows3.at[pl.ds(p * BS, BS), :] for p in range(2)]

        def _zero(i, _):
            r = i // 8
            q = i % 8
            zb[r, pl.ds(q * 16, 16)] = jnp.zeros((16,), _f32)
            return 0
        lax.fori_loop(0, 128 * 8, _zero, 0)

        def _zero2(i, _):
            r = i // 8
            q = i % 8
            rfp[r, pl.ds(q * 16, 16)] = jnp.zeros((16,), _f32)
            return 0
        lax.fori_loop(0, BS * 8, _zero2, 0)
        for q in range(RPT // 128):
            pltpu.sync_copy(zb, acc.at[pl.ds(sid * RPT + q * 128, 128), :])
        plsc.subcore_barrier()

        for slot, scale_qs in passes:
            eth = et_h if slot == 0 else et1_h
            ch = c_h if slot == 0 else c1_h

            def _chunk(ci, _):
                base = pl.multiple_of(wid * EPW + ci * CH, 128)
                rbase = pl.multiple_of(wid * (EPW // BS) + ci * NSB, 8)
                pltpu.sync_copy(src_h.at[pl.ds(base, CH)], srcv)
                pltpu.sync_copy(eth.at[pl.ds(base, CH)], etv)
                pltpu.sync_copy(ch.at[pl.ds(base, CH)], cv)
                pltpu.sync_copy(dst2_h.at[pl.ds(rbase, NSB), :], dstv)

                def _gix(j, _):
                    sl = pl.ds(j * 16, 16)
                    gidx[sl] = etv[sl] * N + srcv[sl]
                    return 0
                lax.fori_loop(0, CH // 16, _gix, 0)

                def _do_batch(b, p):
                    pltpu.make_async_copy(
                        h_h.at[gidx.at[pl.ds(b * BS, BS)]],
                        rbuf[p], gsem[p]).wait()

                    def _scale(g, _):
                        cvec16 = cv[pl.ds(b * BS + g * 16, 16)]
                        for e in range(16):
                            cbr = jnp.full((16,), cvec16[e], _f32)
                            i = p * BS + g * 16 + e
                            io = g * 16 + e
                            for q in scale_qs:
                                sl = pl.ds(q * 16, 16)
                                rfp[io, sl] = rows3[i, sl] * cbr
                        return 0
                    lax.fori_loop(0, BS // 16, _scale, 0)
                    pltpu.sync_copy(rfp, acc.at[dstv.at[b]], add=True)

                pltpu.async_copy(
                    h_h.at[gidx.at[pl.ds(0, BS)]], rbuf[0], gsem[0])

                def _pair(k, _):
                    b0 = k * 2
                    pltpu.async_copy(
                        h_h.at[gidx.at[pl.ds(b0 * BS + BS, BS)]],
                        rbuf[1], gsem[1])
                    _do_batch(b0, 0)

                    @pl.when(k < NSB // 2 - 1)
                    def _():
                        pltpu.async_copy(
                            h_h.at[gidx.at[pl.ds(b0 * BS + 2 * BS, BS)]],
                            rbuf[0], gsem[0])
                    _do_batch(b0 + 1, 1)
                    return 0
                lax.fori_loop(0, NSB // 2, _pair, 0)
                return 0
            lax.fori_loop(0, NCHUNK, _chunk, 0)
        plsc.subcore_barrier()

        for q in range(RPT // 128):
            sl = pl.ds(sid * RPT + q * 128, 128)
            pltpu.sync_copy(acc.at[sl, :], zb)
            pltpu.sync_copy(zb, out_h.at[cid, sl, :])
    return _agg


_agg_l1_main = _make_agg(((0, tuple(range(8))),))
_agg_l1_aug = _make_agg(((1, (0, 1, 2, 3)),))
_agg_l2_main = _make_agg(((0, (0, 1, 2, 3)),))
_agg_l2_aug = _make_agg(((1, (4, 5)),))


@functools.partial(
    pl.kernel,
    out_type=(jax.ShapeDtypeStruct((B, 128), _f32),
              jax.ShapeDtypeStruct((B, 128), _f32),
              jax.ShapeDtypeStruct((B, 128), _f32),
              jax.ShapeDtypeStruct((B, 128), _f32)),
    mesh=_mesh,
    compiler_params=_sc_params,
    scratch_types=[
        pltpu.VMEM((128,), _i32),
        pltpu.VMEM((128, 128), _f32),
        pltpu.SemaphoreType.DMA,
    ],
)
def _pair_kernel(x1_h, x2_h, i0_h, i1_h, e1a_h, e1b_h, e2a_h, e2b_h,
                 iv, buf, sem):
    cid = lax.axis_index("c")
    sid = lax.axis_index("s")
    wid = sid * NC + cid
    base = pl.multiple_of(wid * (B // NW), 128)
    sl = pl.ds(base, 128)
    pltpu.sync_copy(i0_h.at[sl], iv)
    pltpu.async_copy(x1_h.at[iv], buf, sem).wait()
    pltpu.sync_copy(buf, e1a_h.at[sl, :])
    pltpu.async_copy(x2_h.at[iv], buf, sem).wait()
    pltpu.sync_copy(buf, e1b_h.at[sl, :])
    pltpu.sync_copy(i1_h.at[sl], iv)
    pltpu.async_copy(x1_h.at[iv], buf, sem).wait()
    pltpu.sync_copy(buf, e2a_h.at[sl, :])
    pltpu.async_copy(x2_h.at[iv], buf, sem).wait()
    pltpu.sync_copy(buf, e2b_h.at[sl, :])


def _h1_body(xo_ref, xa_ref, w_ref, o_ref):
    o_ref[...] = jnp.concatenate(
        [jnp.dot(xo_ref[...], w_ref[0], preferred_element_type=_f32),
         jnp.dot(xa_ref[...], w_ref[0], preferred_element_type=_f32)],
        axis=1)


_h1_call = pl.pallas_call(
    _h1_body,
    grid=(R,),
    in_specs=[pl.BlockSpec((N, F_IN), lambda r: (0, 0)),
              pl.BlockSpec((N, F_IN), lambda r: (0, 0)),
              pl.BlockSpec((1, F_IN, H1), lambda r: (r, 0, 0))],
    out_specs=pl.BlockSpec((N, 128), lambda r: (r, 0)),
    out_shape=jax.ShapeDtypeStruct((R * N, 128), _f32),
)


def _h2_body(xo_ref, xa_ref, xaa_ref, w_ref, o_ref):
    parts = [jnp.dot(x_ref[:, 0:H1], w_ref[0], preferred_element_type=_f32)
             for x_ref in (xo_ref, xa_ref, xaa_ref)]
    parts.append(jnp.zeros((N, H2), _f32))
    o_ref[...] = jnp.concatenate(parts, axis=1)


_h2_call = pl.pallas_call(
    _h2_body,
    grid=(R,),
    in_specs=[pl.BlockSpec((N, 128), lambda r: (0, 0)),
              pl.BlockSpec((N, 128), lambda r: (0, 0)),
              pl.BlockSpec((N, 128), lambda r: (0, 0)),
              pl.BlockSpec((1, H1, H2), lambda r: (r, 0, 0))],
    out_specs=pl.BlockSpec((N, 128), lambda r: (r, 0)),
    out_shape=jax.ShapeDtypeStruct((R * N, 128), _f32),
)


def _make_combine(F, H, col, relu):
    def body(a_ref, x_ref, root_ref, b_ref, o_ref):
        a = a_ref[0, :, col * H:(col + 1) * H] + a_ref[1, :, col * H:(col + 1) * H]
        y = (a
             + jnp.dot(x_ref[:, 0:F], root_ref[...],
                       preferred_element_type=_f32)
             + b_ref[...])
        if relu:
            y = jnp.maximum(y, 0.0)
        pad = jnp.zeros((y.shape[0], 128 - H), _f32)
        o_ref[...] = jnp.concatenate([y, pad], axis=1)

    BN = 2000
    return pl.pallas_call(
        body,
        grid=(N // BN,),
        in_specs=[pl.BlockSpec((NC, BN, 128), lambda i: (0, i, 0)),
                  pl.BlockSpec((BN, F_IN), lambda i: (i, 0)),
                  pl.BlockSpec((F, H), lambda i: (0, 0)),
                  pl.BlockSpec((1, H), lambda i: (0, 0))],
        out_specs=pl.BlockSpec((BN, 128), lambda i: (i, 0)),
        out_shape=jax.ShapeDtypeStruct((N, 128), _f32),
    )


_comb1_0 = _make_combine(F_IN, H1, 0, True)
_comb1_1 = _make_combine(F_IN, H1, 1, True)
_comb2_0 = _make_combine(H1, H2, 0, False)
_comb2_1 = _make_combine(H1, H2, 1, False)
_comb2_2 = _make_combine(H1, H2, 2, False)


def _epi_body(x2o_ref, x2a_ref, x2aa_ref, e1a_ref, e1b_ref, e2a_ref, e2b_ref,
              attt_ref, wb_ref, bb_ref, wc_ref, bc_ref,
              log_ref, ro_ref, roa_ref, x2_ref):
    x2o = x2o_ref[:, 0:H2]
    x2_ref[...] = x2o
    hos = jax.nn.sigmoid(jnp.mean(x2o, axis=0, keepdims=True))
    wb = wb_ref[...]
    bb = bb_ref[0, 0]
    sc1 = jnp.sum(jnp.dot(x2o, wb, preferred_element_type=_f32) * hos,
                  axis=1, keepdims=True) + bb
    sc2 = jnp.sum(jnp.dot(x2a_ref[:, 0:H2], wb,
                          preferred_element_type=_f32) * hos,
                  axis=1, keepdims=True) + bb
    sc2a = jnp.sum(jnp.dot(x2aa_ref[:, 0:H2], wb,
                           preferred_element_type=_f32) * hos,
                   axis=1, keepdims=True) + bb
    ro_ref[...] = jnp.concatenate([sc1, sc2], axis=1)
    roa_ref[...] = jnp.concatenate([sc1, sc2a], axis=1)
    a0 = attt_ref[0, 0]
    a1 = attt_ref[0, 1]
    log = (a0 * jnp.dot(e1a_ref[:, 0:H1], wc_ref[0:H1, :],
                        preferred_element_type=_f32)
           + a1 * jnp.dot(e1b_ref[:, 0:H2], wc_ref[H1:H1 + H2, :],
                          preferred_element_type=_f32)
           + a0 * jnp.dot(e2a_ref[:, 0:H1], wc_ref[H1 + H2:2 * H1 + H2, :],
                          preferred_element_type=_f32)
           + a1 * jnp.dot(e2b_ref[:, 0:H2], wc_ref[2 * H1 + H2:, :],
                          preferred_element_type=_f32)
           + bc_ref[...])
    log_ref[...] = log


_epi_call = pl.pallas_call(
    _epi_body,
    out_shape=(jax.ShapeDtypeStruct((B, R), _f32),
               jax.ShapeDtypeStruct((N, 2), _f32),
               jax.ShapeDtypeStruct((N, 2), _f32),
               jax.ShapeDtypeStruct((N, H2), _f32)),
)


def kernel(x_o, x_a, edge_index, edge_type, edge_type1, idx, W1, root1, b1,
           W2, root2, b2, attt, Wb, bb, Wc, bc):
    src = edge_index[0]
    dst = edge_index[1]
    pad = E_PAD - E
    zpad = jnp.zeros((pad,), _i32)
    src_p = jnp.concatenate([src, zpad])
    dst_p = jnp.concatenate([dst, jnp.full((pad,), N, _i32)])
    et_p = jnp.concatenate([edge_type, zpad])
    et1_p = jnp.concatenate([edge_type1, zpad])
    dst2 = dst_p.reshape(E_PAD // BS, BS)

    c_all = _prep_kernel(dst_p, et_p, et1_p)
    c_et = c_all[0]
    c_et1 = c_all[1]

    h1 = _h1_call(x_o, x_a, W1)
    agg_oa = _agg_l1_main(h1, src_p, dst2, et_p, et1_p, c_et, c_et1)
    agg_aa = _agg_l1_aug(h1, src_p, dst2, et_p, et1_p, c_et, c_et1)
    b1r = b1.reshape(1, H1)
    x1_o = _comb1_0(agg_oa, x_o, root1, b1r)
    x1_a = _comb1_1(agg_oa, x_a, root1, b1r)
    x1_aa = _comb1_0(agg_aa, x_o, root1, b1r)

    h2 = _h2_call(x1_o, x1_a, x1_aa, W2)
    agg2 = _agg_l2_main(h2, src_p, dst2, et_p, et1_p, c_et, c_et1)
    agg2_aug = _agg_l2_aug(h2, src_p, dst2, et_p, et1_p, c_et, c_et1)
    b2r = b2.reshape(1, H2)
    x2_o = _comb2_0(agg2, x1_o, root2, b2r)
    x2_a = _comb2_1(agg2, x1_a, root2, b2r)
    x2_aa = _comb2_2(agg2_aug, x1_aa, root2, b2r)

    e1a, e1b, e2a, e2b = _pair_kernel(x1_o, x2_o, idx[0], idx[1])
    log, ret_os, ret_os_a, x2_exact = _epi_call(
        x2_o, x2_a, x2_aa, e1a, e1b, e2a, e2b,
        attt.reshape(1, 2), Wb[0], bb.reshape(1, 1), Wc, bc.reshape(1, R))
    return (log, ret_os, ret_os_a, x2_exact)


# speedup vs baseline: 1.2030x; 1.0108x over previous
import functools

import jax
import jax.numpy as jnp
from jax import lax
from jax.experimental import pallas as pl
from jax.experimental.pallas import tpu as pltpu
from jax.experimental.pallas import tpu_sc as plsc

N = 10000
E = 320000
R = 65
F_IN = 128
H1 = 64
H2 = 32
B = 4096

NC = 2
NS = 16
NW = NC * NS
E_PAD = 327680
EPW = E_PAD // NW
EPT = E_PAD // NS
CH = 2048
NB = CH // 128
NCHUNK = EPW // CH
BS = 32
NSB = CH // BS
N_PAD = 10240
NR_PAD = 655360
NR_SL = NR_PAD // NS
ZCH = 5120
RPT = N_PAD // NS

_mesh = plsc.VectorSubcoreMesh(
    core_axis_name="c", subcore_axis_name="s", num_cores=NC, num_subcores=NS)
_sc_params = pltpu.CompilerParams(
    use_tc_tiling_on_sc=False, needs_layout_passes=False)

_f32 = jnp.float32
_bf16 = jnp.bfloat16
_i32 = jnp.int32

def _mk_perm(width):
    perm = [0] * width
    for g in range(width // 32):
        for j in range(16):
            perm[32 * g + 2 * j] = 32 * g + j
            perm[32 * g + 2 * j + 1] = 32 * g + 16 + j
    return perm


_PERM64 = _mk_perm(64)
_PERM32 = _mk_perm(32)


@functools.partial(
    pl.kernel,
    out_type=jax.ShapeDtypeStruct((2, E_PAD), _f32),
    mesh=_mesh,
    compiler_params=_sc_params,
    scratch_types=[
        pltpu.VMEM((CH,), _i32),
        pltpu.VMEM((CH,), _i32),
        pltpu.VMEM((NB, 128), _i32),
        pltpu.VMEM((CH,), _f32),
        pltpu.VMEM((128,), _f32),
        pltpu.VMEM((ZCH,), _f32),
        pltpu.SemaphoreType.DMA,
        pltpu.VMEM_SHARED((NR_PAD,), _f32),
        pltpu.VMEM_SHARED((NR_PAD,), _f32),
    ],
)
def _prep_kernel(dst_h, et_h, et1_h, out_h, dstv, etv, comp, cb, ones,
                 zb, sem, cnt0, cnt1):
    cid = lax.axis_index("c")
    sid = lax.axis_index("s")
    wid = sid * NC + cid

    def _fill(i, _):
        ones[pl.ds(i * 16, 16)] = jnp.ones((16,), _f32)
        return 0
    lax.fori_loop(0, 128 // 16, _fill, 0)

    def _zero(i, _):
        zb[pl.ds(i * 16, 16)] = jnp.zeros((16,), _f32)
        return 0
    lax.fori_loop(0, ZCH // 16, _zero, 0)

    for q in range(NR_SL // ZCH):
        sl = pl.ds(sid * NR_SL + q * ZCH, ZCH)
        pltpu.sync_copy(zb, cnt0.at[sl])
        pltpu.sync_copy(zb, cnt1.at[sl])
    plsc.subcore_barrier()

    def _count_chunk(ci, _):
        base = pl.multiple_of(sid * EPT + ci * CH, 128)
        pltpu.sync_copy(dst_h.at[pl.ds(base, CH)], dstv)
        for eth, cnt_sh in ((et_h, cnt0), (et1_h, cnt1)):
            pltpu.sync_copy(eth.at[pl.ds(base, CH)], etv)

            def _rows(r, _):
                for k in range(8):
                    off = r * 128 + k * 16
                    comp[r, pl.ds(k * 16, 16)] = (
                        dstv[pl.ds(off, 16)] * R + etv[pl.ds(off, 16)])
                return 0
            lax.fori_loop(0, NB, _rows, 0)

            def _scat(b, _):
                pltpu.sync_copy(ones, cnt_sh.at[comp.at[b]], add=True)
                return 0
            lax.fori_loop(0, NB, _scat, 0)
        return 0
    lax.fori_loop(0, EPT // CH, _count_chunk, 0)
    plsc.subcore_barrier()

    for cnt_sh in (cnt0, cnt1):
        for q in range(NR_SL // ZCH):
            sl = pl.ds(sid * NR_SL + q * ZCH, ZCH)
            pltpu.sync_copy(cnt_sh.at[sl], zb)

            def _recip(i, _):
                s2 = pl.ds(i * 16, 16)
                zb[s2] = 1.0 / jnp.maximum(zb[s2], 1.0)
                return 0
            lax.fori_loop(0, ZCH // 16, _recip, 0)
            pltpu.sync_copy(zb, cnt_sh.at[sl])
    plsc.subcore_barrier()

    def _coef_chunk(ci, _):
        base = pl.multiple_of(wid * EPW + ci * CH, 128)
        pltpu.sync_copy(dst_h.at[pl.ds(base, CH)], dstv)
        for k, (eth, cnt_sh) in enumerate(((et_h, cnt0), (et1_h, cnt1))):
            pltpu.sync_copy(eth.at[pl.ds(base, CH)], etv)

            def _rows(r, _):
                for kk in range(8):
                    off = r * 128 + kk * 16
                    comp[r, pl.ds(kk * 16, 16)] = (
                        dstv[pl.ds(off, 16)] * R + etv[pl.ds(off, 16)])
                return 0
            lax.fori_loop(0, NB, _rows, 0)

            def _gath(b, _):
                pltpu.async_copy(cnt_sh.at[comp.at[b]],
                                 cb.at[pl.ds(b * 128, 128)], sem).wait()
                return 0
            lax.fori_loop(0, NB, _gath, 0)
            pltpu.sync_copy(cb, out_h.at[k, pl.ds(base, CH)])
        return 0
    lax.fori_loop(0, NCHUNK, _coef_chunk, 0)


def _make_agg(passes):
    @functools.partial(
        pl.kernel,
        out_type=jax.ShapeDtypeStruct((NC, N_PAD, 128), _f32),
        mesh=_mesh,
        compiler_params=_sc_params,
        scratch_types=[
            pltpu.VMEM((CH,), _i32),
            pltpu.VMEM((CH,), _i32),
            pltpu.VMEM((CH,), _f32),
            pltpu.VMEM((NSB, BS), _i32),
            pltpu.VMEM((CH,), _i32),
            pltpu.VMEM((2 * BS, 128), _f32),
            pltpu.VMEM((BS, 128), _f32),
            pltpu.VMEM((128, 128), _f32),
            pltpu.SemaphoreType.DMA,
            pltpu.SemaphoreType.DMA,
            pltpu.VMEM_SHARED((N_PAD, 128), _f32),
        ],
    )
    def _agg(h_h, src_h, dst2_h, et_h, et1_h, c_h, c1_h, out_h,
             srcv, etv, cv, dstv, gidx, rows3, rfp, zb,
             gs0, gs1, acc):
        cid = lax.axis_index("c")
        sid = lax.axis_index("s")
        wid = sid * NC + cid
        gsem = (gs0, gs1)
        rbuf = [rows3.at[pl.ds(p * BS, BS), :] for p in range(2)]

        def _zero(i, _):
            r = i // 8
            q = i % 8
            zb[r, pl.ds(q * 16, 16)] = jnp.zeros((16,), _f32)
            return 0
        lax.fori_loop(0, 128 * 8, _zero, 0)

        def _zero2(i, _):
            r = i // 8
            q = i % 8
            rfp[r, pl.ds(q * 16, 16)] = jnp.zeros((16,), _f32)
            return 0
        lax.fori_loop(0, BS * 8, _zero2, 0)
        for q in range(RPT // 128):
            pltpu.sync_copy(zb, acc.at[pl.ds(sid * RPT + q * 128, 128), :])
        plsc.subcore_barrier()

        for slot, scale_qs in passes:
            eth = et_h if slot == 0 else et1_h
            ch = c_h if slot == 0 else c1_h

            def _chunk(ci, _):
                base = pl.multiple_of(wid * EPW + ci * CH, 128)
                rbase = pl.multiple_of(wid * (EPW // BS) + ci * NSB, 8)
                pltpu.sync_copy(src_h.at[pl.ds(base, CH)], srcv)
                pltpu.sync_copy(eth.at[pl.ds(base, CH)], etv)
                pltpu.sync_copy(ch.at[pl.ds(base, CH)], cv)
                pltpu.sync_copy(dst2_h.at[pl.ds(rbase, NSB), :], dstv)

                def _gix(j, _):
                    sl = pl.ds(j * 16, 16)
                    gidx[sl] = etv[sl] * N + srcv[sl]
                    return 0
                lax.fori_loop(0, CH // 16, _gix, 0)

                def _do_batch(b, p):
                    pltpu.make_async_copy(
                        h_h.at[gidx.at[pl.ds(b * BS, BS)]],
                        rbuf[p], gsem[p]).wait()

                    def _scale(g, _):
                        cvec16 = cv[pl.ds(b * BS + g * 16, 16)]
                        for e in range(16):
                            cbr = jnp.full((16,), cvec16[e], _f32)
                            i = p * BS + g * 16 + e
                            io = g * 16 + e
                            for q in scale_qs:
                                sl = pl.ds(q * 16, 16)
                                rfp[io, sl] = rows3[i, sl] * cbr
                        return 0
                    lax.fori_loop(0, BS // 16, _scale, 0)
                    pltpu.sync_copy(rfp, acc.at[dstv.at[b]], add=True)

                pltpu.async_copy(
                    h_h.at[gidx.at[pl.ds(0, BS)]], rbuf[0], gsem[0])

                def _pair(k, _):
                    b0 = k * 2
                    pltpu.async_copy(
                        h_h.at[gidx.at[pl.ds(b0 * BS + BS, BS)]],
                        rbuf[1], gsem[1])
                    _do_batch(b0, 0)

                    @pl.when(k < NSB // 2 - 1)
                    def _():
                        pltpu.async_copy(
                            h_h.at[gidx.at[pl.ds(b0 * BS + 2 * BS, BS)]],
                            rbuf[0], gsem[0])
                    _do_batch(b0 + 1, 1)
                    return 0
                lax.fori_loop(0, NSB // 2, _pair, 0)
                return 0
            lax.fori_loop(0, NCHUNK, _chunk, 0)
        plsc.subcore_barrier()

        for q in range(RPT // 128):
            sl = pl.ds(sid * RPT + q * 128, 128)
            pltpu.sync_copy(acc.at[sl, :], zb)
            pltpu.sync_copy(zb, out_h.at[cid, sl, :])
    return _agg


_agg_l1_main = _make_agg(((0, tuple(range(8))),))
_agg_l1_aug = _make_agg(((1, (0, 1, 2, 3)),))
_agg_l2_main = _make_agg(((0, (0, 1, 2, 3)),))
_agg_l2_aug = _make_agg(((1, (0, 1)),))


@functools.partial(
    pl.kernel,
    out_type=(jax.ShapeDtypeStruct((B, 128), _f32),
              jax.ShapeDtypeStruct((B, 128), _f32),
              jax.ShapeDtypeStruct((B, 128), _f32),
              jax.ShapeDtypeStruct((B, 128), _f32)),
    mesh=_mesh,
    compiler_params=_sc_params,
    scratch_types=[
        pltpu.VMEM((128,), _i32),
        pltpu.VMEM((128, 128), _f32),
        pltpu.SemaphoreType.DMA,
    ],
)
def _pair_kernel(x1_h, x2_h, i0_h, i1_h, e1a_h, e1b_h, e2a_h, e2b_h,
                 iv, buf, sem):
    cid = lax.axis_index("c")
    sid = lax.axis_index("s")
    wid = sid * NC + cid
    base = pl.multiple_of(wid * (B // NW), 128)
    sl = pl.ds(base, 128)
    pltpu.sync_copy(i0_h.at[sl], iv)
    pltpu.async_copy(x1_h.at[iv], buf, sem).wait()
    pltpu.sync_copy(buf, e1a_h.at[sl, :])
    pltpu.async_copy(x2_h.at[iv], buf, sem).wait()
    pltpu.sync_copy(buf, e1b_h.at[sl, :])
    pltpu.sync_copy(i1_h.at[sl], iv)
    pltpu.async_copy(x1_h.at[iv], buf, sem).wait()
    pltpu.sync_copy(buf, e2a_h.at[sl, :])
    pltpu.async_copy(x2_h.at[iv], buf, sem).wait()
    pltpu.sync_copy(buf, e2b_h.at[sl, :])


def _h1_body(xo_ref, xa_ref, w_ref, o_ref):
    o_ref[...] = jnp.concatenate(
        [jnp.dot(xo_ref[...], w_ref[0], preferred_element_type=_f32),
         jnp.dot(xa_ref[...], w_ref[0], preferred_element_type=_f32)],
        axis=1)


_h1_call = pl.pallas_call(
    _h1_body,
    grid=(R,),
    in_specs=[pl.BlockSpec((N, F_IN), lambda r: (0, 0)),
              pl.BlockSpec((N, F_IN), lambda r: (0, 0)),
              pl.BlockSpec((1, F_IN, H1), lambda r: (r, 0, 0))],
    out_specs=pl.BlockSpec((N, 128), lambda r: (r, 0)),
    out_shape=jax.ShapeDtypeStruct((R * N, 128), _f32),
)


def _h2m_body(xo_ref, xa_ref, w_ref, o_ref):
    parts = [jnp.dot(x_ref[:, 0:H1], w_ref[0], preferred_element_type=_f32)
             for x_ref in (xo_ref, xa_ref)]
    parts.append(jnp.zeros((N, H1), _f32))
    o_ref[...] = jnp.concatenate(parts, axis=1)


_h2m_call = pl.pallas_call(
    _h2m_body,
    grid=(R,),
    in_specs=[pl.BlockSpec((N, 128), lambda r: (0, 0)),
              pl.BlockSpec((N, 128), lambda r: (0, 0)),
              pl.BlockSpec((1, H1, H2), lambda r: (r, 0, 0))],
    out_specs=pl.BlockSpec((N, 128), lambda r: (r, 0)),
    out_shape=jax.ShapeDtypeStruct((R * N, 128), _f32),
)


def _h2a_body(xaa_ref, w_ref, o_ref):
    y = jnp.dot(xaa_ref[:, 0:H1], w_ref[0], preferred_element_type=_f32)
    o_ref[...] = jnp.concatenate([y, jnp.zeros((N, 128 - H2), _f32)], axis=1)


_h2a_call = pl.pallas_call(
    _h2a_body,
    grid=(R,),
    in_specs=[pl.BlockSpec((N, 128), lambda r: (0, 0)),
              pl.BlockSpec((1, H1, H2), lambda r: (r, 0, 0))],
    out_specs=pl.BlockSpec((N, 128), lambda r: (r, 0)),
    out_shape=jax.ShapeDtypeStruct((R * N, 128), _f32),
)


def _make_combine(F, H, col, relu):
    def body(a_ref, x_ref, root_ref, b_ref, o_ref):
        a = a_ref[0, :, col * H:(col + 1) * H] + a_ref[1, :, col * H:(col + 1) * H]
        y = (a
             + jnp.dot(x_ref[:, 0:F], root_ref[...],
                       preferred_element_type=_f32)
             + b_ref[...])
        if relu:
            y = jnp.maximum(y, 0.0)
        pad = jnp.zeros((y.shape[0], 128 - H), _f32)
        o_ref[...] = jnp.concatenate([y, pad], axis=1)

    BN = 2000
    return pl.pallas_call(
        body,
        grid=(N // BN,),
        in_specs=[pl.BlockSpec((NC, BN, 128), lambda i: (0, i, 0)),
                  pl.BlockSpec((BN, F_IN), lambda i: (i, 0)),
                  pl.BlockSpec((F, H), lambda i: (0, 0)),
                  pl.BlockSpec((1, H), lambda i: (0, 0))],
        out_specs=pl.BlockSpec((BN, 128), lambda i: (i, 0)),
        out_shape=jax.ShapeDtypeStruct((N, 128), _f32),
    )


_comb1_0 = _make_combine(F_IN, H1, 0, True)
_comb1_1 = _make_combine(F_IN, H1, 1, True)
_comb2_0 = _make_combine(H1, H2, 0, False)
_comb2_1 = _make_combine(H1, H2, 1, False)
_comb2_2 = _make_combine(H1, H2, 2, False)


def _epi_body(x2o_ref, x2a_ref, x2aa_ref, e1a_ref, e1b_ref, e2a_ref, e2b_ref,
              attt_ref, wb_ref, bb_ref, wc_ref, bc_ref,
              log_ref, ro_ref, roa_ref, x2_ref):
    x2o = x2o_ref[:, 0:H2]
    x2_ref[...] = x2o
    hos = jax.nn.sigmoid(jnp.mean(x2o, axis=0, keepdims=True))
    wb = wb_ref[...]
    bb = bb_ref[0, 0]
    sc1 = jnp.sum(jnp.dot(x2o, wb, preferred_element_type=_f32) * hos,
                  axis=1, keepdims=True) + bb
    sc2 = jnp.sum(jnp.dot(x2a_ref[:, 0:H2], wb,
                          preferred_element_type=_f32) * hos,
                  axis=1, keepdims=True) + bb
    sc2a = jnp.sum(jnp.dot(x2aa_ref[:, 0:H2], wb,
                           preferred_element_type=_f32) * hos,
                   axis=1, keepdims=True) + bb
    ro_ref[...] = jnp.concatenate([sc1, sc2], axis=1)
    roa_ref[...] = jnp.concatenate([sc1, sc2a], axis=1)
    a0 = attt_ref[0, 0]
    a1 = attt_ref[0, 1]
    log = (a0 * jnp.dot(e1a_ref[:, 0:H1], wc_ref[0:H1, :],
                        preferred_element_type=_f32)
           + a1 * jnp.dot(e1b_ref[:, 0:H2], wc_ref[H1:H1 + H2, :],
                          preferred_element_type=_f32)
           + a0 * jnp.dot(e2a_ref[:, 0:H1], wc_ref[H1 + H2:2 * H1 + H2, :],
                          preferred_element_type=_f32)
           + a1 * jnp.dot(e2b_ref[:, 0:H2], wc_ref[2 * H1 + H2:, :],
                          preferred_element_type=_f32)
           + bc_ref[...])
    log_ref[...] = log


_epi_call = pl.pallas_call(
    _epi_body,
    out_shape=(jax.ShapeDtypeStruct((B, R), _f32),
               jax.ShapeDtypeStruct((N, 2), _f32),
               jax.ShapeDtypeStruct((N, 2), _f32),
               jax.ShapeDtypeStruct((N, H2), _f32)),
)


def kernel(x_o, x_a, edge_index, edge_type, edge_type1, idx, W1, root1, b1,
           W2, root2, b2, attt, Wb, bb, Wc, bc):
    src = edge_index[0]
    dst = edge_index[1]
    pad = E_PAD - E
    zpad = jnp.zeros((pad,), _i32)
    src_p = jnp.concatenate([src, zpad])
    dst_p = jnp.concatenate([dst, jnp.full((pad,), N, _i32)])
    et_p = jnp.concatenate([edge_type, zpad])
    et1_p = jnp.concatenate([edge_type1, zpad])
    dst2 = dst_p.reshape(E_PAD // BS, BS)

    c_all = _prep_kernel(dst_p, et_p, et1_p)
    c_et = c_all[0]
    c_et1 = c_all[1]

    h1 = _h1_call(x_o, x_a, W1)
    agg_oa = _agg_l1_main(h1, src_p, dst2, et_p, et1_p, c_et, c_et1)
    agg_aa = _agg_l1_aug(h1, src_p, dst2, et_p, et1_p, c_et, c_et1)
    b1r = b1.reshape(1, H1)
    x1_o = _comb1_0(agg_oa, x_o, root1, b1r)
    x1_a = _comb1_1(agg_oa, x_a, root1, b1r)
    x1_aa = _comb1_0(agg_aa, x_o, root1, b1r)

    h2m = _h2m_call(x1_o, x1_a, W2)
    h2a = _h2a_call(x1_aa, W2)
    agg2 = _agg_l2_main(h2m, src_p, dst2, et_p, et1_p, c_et, c_et1)
    agg2_aug = _agg_l2_aug(h2a, src_p, dst2, et_p, et1_p, c_et, c_et1)
    b2r = b2.reshape(1, H2)
    x2_o = _comb2_0(agg2, x1_o, root2, b2r)
    x2_a = _comb2_1(agg2, x1_a, root2, b2r)
    x2_aa = _comb2_0(agg2_aug, x1_aa, root2, b2r)

    e1a, e1b, e2a, e2b = _pair_kernel(x1_o, x2_o, idx[0], idx[1])
    log, ret_os, ret_os_a, x2_exact = _epi_call(
        x2_o, x2_a, x2_aa, e1a, e1b, e2a, e2b,
        attt.reshape(1, 2), Wb[0], bb.reshape(1, 1), Wc, bc.reshape(1, R))
    return (log, ret_os, ret_os_a, x2_exact)
